# Initial kernel scaffold; baseline (speedup 1.0000x reference)
import functools

import jax
import jax.numpy as jnp
from jax import lax
from jax.experimental import pallas as pl
from jax.experimental.pallas import tpu as pltpu
from jax.experimental.pallas import tpu_sc as plsc

NN = 10000
EE = 320000
D = 128
L = 16

NC = 2
NS = 16
NW = NC * NS
EPW = EE // NW
K = 80
NCHUNK = EPW // K
RCH = 80
NRC = NN // RCH
ZITER = (NRC + NS - 1) // NS


def _sc_edge_body(a_hbm, b_hbm, c_hbm, src_hbm, dst_hbm, out_hbm,
                  srcv, dstv, av, bv, mv, accum, sema, semb, semc):
    c = lax.axis_index("c")
    s = lax.axis_index("s")
    wid = s * NC + c

    zvec = jnp.zeros((L,), jnp.float32)

    def zrow(r, carry):
        for g in range(D // L):
            av[r, pl.ds(g * L, L)] = zvec
        return carry
    lax.fori_loop(0, RCH, zrow, 0)

    def zchunk(j, carry):
        ch = s + j * NS

        @pl.when(ch < NRC)
        def _():
            pltpu.sync_copy(av, accum.at[pl.ds(ch * RCH, RCH)])
        return carry
    lax.fori_loop(0, ZITER, zchunk, 0)

    plsc.subcore_barrier()

    def echunk(t, carry):
        base = wid * EPW + t * K
        pltpu.sync_copy(src_hbm.at[pl.ds(base, K)], srcv)
        pltpu.sync_copy(dst_hbm.at[pl.ds(base, K)], dstv)
        ca = pltpu.async_copy(a_hbm.at[srcv], av, sema)
        cb = pltpu.async_copy(b_hbm.at[dstv], bv, semb)
        cc = pltpu.async_copy(c_hbm.at[pl.ds(base, K)], mv, semc)
        ca.wait()
        cb.wait()
        cc.wait()

        def crow(r, inner):
            for g in range(D // L):
                sl = pl.ds(g * L, L)
                v = av[r, sl] + bv[r, sl] + mv[r, sl]
                mv[r, sl] = jnp.maximum(v, 0.0)
            return inner
        lax.fori_loop(0, K, crow, 0)

        pltpu.sync_copy(mv, accum.at[dstv], add=True)
        return carry
    lax.fori_loop(0, NCHUNK, echunk, 0)

    plsc.subcore_barrier()

    def wchunk(j, carry):
        ch = s + j * NS

        @pl.when(ch < NRC)
        def _():
            pltpu.sync_copy(accum.at[pl.ds(ch * RCH, RCH)],
                            out_hbm.at[c, pl.ds(ch * RCH, RCH)])
        return carry
    lax.fori_loop(0, ZITER, wchunk, 0)


_sc_edge = functools.partial(
    pl.kernel,
    mesh=plsc.VectorSubcoreMesh(core_axis_name="c", subcore_axis_name="s"),
    out_type=jax.ShapeDtypeStruct((NC, NN, D), jnp.float32),
    scratch_types=[
        pltpu.VMEM((K,), jnp.int32),
        pltpu.VMEM((K,), jnp.int32),
        pltpu.VMEM((K, D), jnp.float32),
        pltpu.VMEM((K, D), jnp.float32),
        pltpu.VMEM((K, D), jnp.float32),
        pltpu.VMEM_SHARED((NN, D), jnp.float32),
        pltpu.SemaphoreType.DMA,
        pltpu.SemaphoreType.DMA,
        pltpu.SemaphoreType.DMA,
    ],
)(_sc_edge_body)


BN = 1000
BEDG = 4000


def _dot(a, b):
    return jnp.dot(a, b, preferred_element_type=jnp.float32)


def _proj_body(h_ref, ws_ref, wd_ref, bm_ref, a_ref, b_ref):
    h = h_ref[...]
    a_ref[...] = _dot(h, ws_ref[...])
    b_ref[...] = _dot(h, wd_ref[...]) + bm_ref[...]


_proj = pl.pallas_call(
    _proj_body,
    grid=(NN // BN,),
    in_specs=[
        pl.BlockSpec((BN, D), lambda i: (i, 0)),
        pl.BlockSpec((D, D), lambda i: (0, 0)),
        pl.BlockSpec((D, D), lambda i: (0, 0)),
        pl.BlockSpec((1, D), lambda i: (0, 0)),
    ],
    out_specs=[pl.BlockSpec((BN, D), lambda i: (i, 0)),
               pl.BlockSpec((BN, D), lambda i: (i, 0))],
    out_shape=[jax.ShapeDtypeStruct((NN, D), jnp.float32),
               jax.ShapeDtypeStruct((NN, D), jnp.float32)],
)


def _edgeproj_body(ea_ref, w0_ref, w1_ref, c0_ref, c1_ref):
    ea = ea_ref[...]
    c0_ref[...] = _dot(ea, w0_ref[...])
    c1_ref[...] = _dot(ea, w1_ref[...])


_edgeproj = pl.pallas_call(
    _edgeproj_body,
    grid=(EE // BEDG,),
    in_specs=[
        pl.BlockSpec((BEDG, 16), lambda i: (i, 0)),
        pl.BlockSpec((16, D), lambda i: (0, 0)),
        pl.BlockSpec((16, D), lambda i: (0, 0)),
    ],
    out_specs=[pl.BlockSpec((BEDG, D), lambda i: (i, 0)),
               pl.BlockSpec((BEDG, D), lambda i: (i, 0))],
    out_shape=[jax.ShapeDtypeStruct((EE, D), jnp.float32),
               jax.ShapeDtypeStruct((EE, D), jnp.float32)],
)


def _upd_body(h_ref, p_ref, wuh_ref, wua_ref, bu_ref, ws_ref, wd_ref, bm_ref,
              h1_ref, a1_ref, b1_ref):
    agg = p_ref[0] + p_ref[1]
    h1 = jnp.maximum(
        _dot(h_ref[...], wuh_ref[...]) + _dot(agg, wua_ref[...]) + bu_ref[...],
        0.0)
    h1_ref[...] = h1
    a1_ref[...] = _dot(h1, ws_ref[...])
    b1_ref[...] = _dot(h1, wd_ref[...]) + bm_ref[...]


_upd = pl.pallas_call(
    _upd_body,
    grid=(NN // BN,),
    in_specs=[
        pl.BlockSpec((BN, D), lambda i: (i, 0)),
        pl.BlockSpec((NC, BN, D), lambda i: (0, i, 0)),
        pl.BlockSpec((D, D), lambda i: (0, 0)),
        pl.BlockSpec((D, D), lambda i: (0, 0)),
        pl.BlockSpec((1, D), lambda i: (0, 0)),
        pl.BlockSpec((D, D), lambda i: (0, 0)),
        pl.BlockSpec((D, D), lambda i: (0, 0)),
        pl.BlockSpec((1, D), lambda i: (0, 0)),
    ],
    out_specs=[pl.BlockSpec((BN, D), lambda i: (i, 0)),
               pl.BlockSpec((BN, D), lambda i: (i, 0)),
               pl.BlockSpec((BN, D), lambda i: (i, 0))],
    out_shape=[jax.ShapeDtypeStruct((NN, D), jnp.float32),
               jax.ShapeDtypeStruct((NN, D), jnp.float32),
               jax.ShapeDtypeStruct((NN, D), jnp.float32)],
)


def _fin_body(h_ref, p_ref, wuh_ref, wua_ref, bu_ref, o_ref):
    agg = p_ref[0] + p_ref[1]
    o_ref[...] = (_dot(h_ref[...], wuh_ref[...]) + _dot(agg, wua_ref[...])
                  + bu_ref[...])


_fin = pl.pallas_call(
    _fin_body,
    grid=(NN // BN,),
    in_specs=[
        pl.BlockSpec((BN, D), lambda i: (i, 0)),
        pl.BlockSpec((NC, BN, D), lambda i: (0, i, 0)),
        pl.BlockSpec((D, D), lambda i: (0, 0)),
        pl.BlockSpec((D, D), lambda i: (0, 0)),
        pl.BlockSpec((1, D), lambda i: (0, 0)),
    ],
    out_specs=pl.BlockSpec((BN, D), lambda i: (i, 0)),
    out_shape=jax.ShapeDtypeStruct((NN, D), jnp.float32),
)


def kernel(x, edge_index, edge_attr, Wm0, bm0, Wu0, bu0, Wm1, bm1, Wu1, bu1):
    h0 = x[:, :, 0]
    src = edge_index[0]
    dst = edge_index[1]

    bm0r = bm0.reshape(1, D)
    bm1r = bm1.reshape(1, D)
    bu0r = bu0.reshape(1, D)
    bu1r = bu1.reshape(1, D)

    A0, B0 = _proj(h0, Wm0[:D], Wm0[D:2 * D], bm0r)
    C0, C1 = _edgeproj(edge_attr, Wm0[2 * D:], Wm1[2 * D:])
    prt0 = _sc_edge(A0, B0, C0, src, dst)
    h1, A1, B1 = _upd(h0, prt0, Wu0[:D], Wu0[D:], bu0r,
                      Wm1[:D], Wm1[D:2 * D], bm1r)
    prt1 = _sc_edge(A1, B1, C1, src, dst)
    out = _fin(h1, prt1, Wu1[:D], Wu1[D:], bu1r)
    return out[:, :, None]


# speedup vs baseline: 3.3111x; 3.3111x over previous
import functools

import jax
import jax.numpy as jnp
from jax import lax
from jax.experimental import pallas as pl
from jax.experimental.pallas import tpu as pltpu
from jax.experimental.pallas import tpu_sc as plsc

NN = 10000
EE = 320000
D = 128
L = 16

NC = 2
NS = 16
NW = NC * NS
EPW = EE // NW
K = 80
NCHUNK = EPW // K
RCH = 80
NRC = NN // RCH
ZITER = (NRC + NS - 1) // NS


def _sc_edge_body(a_hbm, b_hbm, c_hbm, src_hbm, dst_hbm, out_hbm,
                  srcv, dstv, av, bv, mv, accum, sema, semb, semc):
    c = lax.axis_index("c")
    s = lax.axis_index("s")
    wid = s * NC + c

    zvec = jnp.zeros((L,), jnp.float32)

    def zrow(r, carry):
        for g in range(D // L):
            av[r, pl.ds(g * L, L)] = zvec
        return carry
    lax.fori_loop(0, RCH, zrow, 0)

    def zchunk(j, carry):
        ch = s + j * NS

        @pl.when(ch < NRC)
        def _():
            pltpu.sync_copy(av, accum.at[pl.ds(ch * RCH, RCH)])
        return carry
    lax.fori_loop(0, ZITER, zchunk, 0)

    plsc.subcore_barrier()

    def echunk(t, carry):
        base = wid * EPW + t * K
        pltpu.sync_copy(src_hbm.at[pl.ds(base, K)], srcv)
        pltpu.sync_copy(dst_hbm.at[pl.ds(base, K)], dstv)
        ca = pltpu.async_copy(a_hbm.at[srcv], av, sema)
        cb = pltpu.async_copy(b_hbm.at[dstv], bv, semb)
        cc = pltpu.async_copy(c_hbm.at[pl.ds(base, K)], mv, semc)
        ca.wait()
        cb.wait()
        cc.wait()

        def crow(r, inner):
            for g in range(D // L):
                sl = pl.ds(g * L, L)
                v = av[r, sl] + bv[r, sl] + mv[r, sl]
                mv[r, sl] = jnp.maximum(v, 0.0)
            return inner
        lax.fori_loop(0, K, crow, 0)

        pltpu.sync_copy(mv, accum.at[dstv], add=True)
        return carry
    lax.fori_loop(0, NCHUNK, echunk, 0)

    plsc.subcore_barrier()

    def wchunk(j, carry):
        ch = s + j * NS

        @pl.when(ch < NRC)
        def _():
            pltpu.sync_copy(accum.at[pl.ds(ch * RCH, RCH)],
                            out_hbm.at[c, pl.ds(ch * RCH, RCH)])
        return carry
    lax.fori_loop(0, ZITER, wchunk, 0)


@functools.cache
def _get_sc_edge():
    return functools.partial(
        pl.kernel,
        mesh=plsc.VectorSubcoreMesh(core_axis_name="c", subcore_axis_name="s",
                                    num_cores=NC, num_subcores=NS),
        out_type=jax.ShapeDtypeStruct((NC, NN, D), jnp.float32),
        scratch_types=[
            pltpu.VMEM((K,), jnp.int32),
            pltpu.VMEM((K,), jnp.int32),
            pltpu.VMEM((K, D), jnp.float32),
            pltpu.VMEM((K, D), jnp.float32),
            pltpu.VMEM((K, D), jnp.float32),
            pltpu.VMEM_SHARED((NN, D), jnp.float32),
            pltpu.SemaphoreType.DMA,
            pltpu.SemaphoreType.DMA,
            pltpu.SemaphoreType.DMA,
        ],
    )(_sc_edge_body)


BN = 1000
BEDG = 4000


def _dot(a, b):
    return jnp.dot(a, b, preferred_element_type=jnp.float32)


def _proj_body(h_ref, ws_ref, wd_ref, bm_ref, a_ref, b_ref):
    h = h_ref[...]
    a_ref[...] = _dot(h, ws_ref[...])
    b_ref[...] = _dot(h, wd_ref[...]) + bm_ref[...]


_proj = pl.pallas_call(
    _proj_body,
    grid=(NN // BN,),
    in_specs=[
        pl.BlockSpec((BN, D), lambda i: (i, 0)),
        pl.BlockSpec((D, D), lambda i: (0, 0)),
        pl.BlockSpec((D, D), lambda i: (0, 0)),
        pl.BlockSpec((1, D), lambda i: (0, 0)),
    ],
    out_specs=[pl.BlockSpec((BN, D), lambda i: (i, 0)),
               pl.BlockSpec((BN, D), lambda i: (i, 0))],
    out_shape=[jax.ShapeDtypeStruct((NN, D), jnp.float32),
               jax.ShapeDtypeStruct((NN, D), jnp.float32)],
)


def _edgeproj_body(ea_ref, w0_ref, w1_ref, c0_ref, c1_ref):
    ea = ea_ref[...]
    c0_ref[...] = _dot(ea, w0_ref[...])
    c1_ref[...] = _dot(ea, w1_ref[...])


_edgeproj = pl.pallas_call(
    _edgeproj_body,
    grid=(EE // BEDG,),
    in_specs=[
        pl.BlockSpec((BEDG, 16), lambda i: (i, 0)),
        pl.BlockSpec((16, D), lambda i: (0, 0)),
        pl.BlockSpec((16, D), lambda i: (0, 0)),
    ],
    out_specs=[pl.BlockSpec((BEDG, D), lambda i: (i, 0)),
               pl.BlockSpec((BEDG, D), lambda i: (i, 0))],
    out_shape=[jax.ShapeDtypeStruct((EE, D), jnp.float32),
               jax.ShapeDtypeStruct((EE, D), jnp.float32)],
)


def _upd_body(h_ref, p_ref, wuh_ref, wua_ref, bu_ref, ws_ref, wd_ref, bm_ref,
              h1_ref, a1_ref, b1_ref):
    agg = p_ref[0] + p_ref[1]
    h1 = jnp.maximum(
        _dot(h_ref[...], wuh_ref[...]) + _dot(agg, wua_ref[...]) + bu_ref[...],
        0.0)
    h1_ref[...] = h1
    a1_ref[...] = _dot(h1, ws_ref[...])
    b1_ref[...] = _dot(h1, wd_ref[...]) + bm_ref[...]


_upd = pl.pallas_call(
    _upd_body,
    grid=(NN // BN,),
    in_specs=[
        pl.BlockSpec((BN, D), lambda i: (i, 0)),
        pl.BlockSpec((NC, BN, D), lambda i: (0, i, 0)),
        pl.BlockSpec((D, D), lambda i: (0, 0)),
        pl.BlockSpec((D, D), lambda i: (0, 0)),
        pl.BlockSpec((1, D), lambda i: (0, 0)),
        pl.BlockSpec((D, D), lambda i: (0, 0)),
        pl.BlockSpec((D, D), lambda i: (0, 0)),
        pl.BlockSpec((1, D), lambda i: (0, 0)),
    ],
    out_specs=[pl.BlockSpec((BN, D), lambda i: (i, 0)),
               pl.BlockSpec((BN, D), lambda i: (i, 0)),
               pl.BlockSpec((BN, D), lambda i: (i, 0))],
    out_shape=[jax.ShapeDtypeStruct((NN, D), jnp.float32),
               jax.ShapeDtypeStruct((NN, D), jnp.float32),
               jax.ShapeDtypeStruct((NN, D), jnp.float32)],
)


def _fin_body(h_ref, p_ref, wuh_ref, wua_ref, bu_ref, o_ref):
    agg = p_ref[0] + p_ref[1]
    o_ref[...] = (_dot(h_ref[...], wuh_ref[...]) + _dot(agg, wua_ref[...])
                  + bu_ref[...])


_fin = pl.pallas_call(
    _fin_body,
    grid=(NN // BN,),
    in_specs=[
        pl.BlockSpec((BN, D), lambda i: (i, 0)),
        pl.BlockSpec((NC, BN, D), lambda i: (0, i, 0)),
        pl.BlockSpec((D, D), lambda i: (0, 0)),
        pl.BlockSpec((D, D), lambda i: (0, 0)),
        pl.BlockSpec((1, D), lambda i: (0, 0)),
    ],
    out_specs=pl.BlockSpec((BN, D), lambda i: (i, 0)),
    out_shape=jax.ShapeDtypeStruct((NN, D), jnp.float32),
)


def kernel(x, edge_index, edge_attr, Wm0, bm0, Wu0, bu0, Wm1, bm1, Wu1, bu1):
    h0 = x[:, :, 0]
    src = edge_index[0]
    dst = edge_index[1]

    bm0r = bm0.reshape(1, D)
    bm1r = bm1.reshape(1, D)
    bu0r = bu0.reshape(1, D)
    bu1r = bu1.reshape(1, D)

    _sc_edge = _get_sc_edge()
    A0, B0 = _proj(h0, Wm0[:D], Wm0[D:2 * D], bm0r)
    C0, C1 = _edgeproj(edge_attr, Wm0[2 * D:], Wm1[2 * D:])
    prt0 = _sc_edge(A0, B0, C0, src, dst)
    h1, A1, B1 = _upd(h0, prt0, Wu0[:D], Wu0[D:], bu0r,
                      Wm1[:D], Wm1[D:2 * D], bm1r)
    prt1 = _sc_edge(A1, B1, C1, src, dst)
    out = _fin(h1, prt1, Wu1[:D], Wu1[D:], bu1r)
    return out[:, :, None]


# speedup vs baseline: 4.4192x; 1.3346x over previous
import functools

import jax
import jax.numpy as jnp
from jax import lax
from jax.experimental import pallas as pl
from jax.experimental.pallas import tpu as pltpu
from jax.experimental.pallas import tpu_sc as plsc

NN = 10000
EE = 320000
D = 128
L = 16

NC = 2
NS = 16
NW = NC * NS
EPW = EE // NW
K = 40
NCHUNK = EPW // K
RCH = 40
NRC = NN // RCH
ZITER = (NRC + NS - 1) // NS


def _sc_edge_body(a_hbm, b_hbm, c_hbm, src_hbm, dst_hbm, out_hbm,
                  si0, si1, di0, di1, av0, av1, bv0, bv1, mv0, mv1, accum,
                  ssi0, ssi1, sdi0, sdi1, sa0, sa1, sb0, sb1, sc0, sc1):
    sis = (si0, si1)
    dis = (di0, di1)
    avs = (av0, av1)
    bvs = (bv0, bv1)
    mvs = (mv0, mv1)
    ssis = (ssi0, ssi1)
    sdis = (sdi0, sdi1)
    sas = (sa0, sa1)
    sbs = (sb0, sb1)
    scs = (sc0, sc1)

    c = lax.axis_index("c")
    s = lax.axis_index("s")
    wid = s * NC + c
    ebase = wid * EPW

    zvec = jnp.zeros((L,), jnp.float32)

    def zrow(r, carry):
        for g in range(D // L):
            av0[r, pl.ds(g * L, L)] = zvec
        return carry
    lax.fori_loop(0, RCH, zrow, 0)

    def zchunk(j, carry):
        ch = s + j * NS

        @pl.when(ch < NRC)
        def _():
            pltpu.sync_copy(av0, accum.at[pl.ds(ch * RCH, RCH)])
        return carry
    lax.fori_loop(0, ZITER, zchunk, 0)

    plsc.subcore_barrier()

    def start_idx(ch, b):
        pltpu.async_copy(src_hbm.at[wid, ch], sis[b], ssis[b])
        pltpu.async_copy(dst_hbm.at[wid, ch], dis[b], sdis[b])

    def wait_idx(b):
        pltpu.make_async_copy(src_hbm.at[0, 0], sis[b], ssis[b]).wait()
        pltpu.make_async_copy(dst_hbm.at[0, 0], dis[b], sdis[b]).wait()

    def start_rows(ch, b):
        pltpu.async_copy(a_hbm.at[sis[b]], avs[b], sas[b])
        pltpu.async_copy(b_hbm.at[dis[b]], bvs[b], sbs[b])
        pltpu.async_copy(c_hbm.at[pl.ds(ebase + ch * K, K)], mvs[b], scs[b])

    def wait_rows(b):
        pltpu.make_async_copy(a_hbm.at[pl.ds(0, K)], avs[b], sas[b]).wait()
        pltpu.make_async_copy(b_hbm.at[pl.ds(0, K)], bvs[b], sbs[b]).wait()
        pltpu.make_async_copy(c_hbm.at[pl.ds(0, K)], mvs[b], scs[b]).wait()

    start_idx(0, 0)
    start_idx(1, 1)
    wait_idx(0)
    start_rows(0, 0)

    def epair(t2, carry):
        t = t2 * 2
        for b in range(2):
            ch = t + b
            wait_rows(b)

            @pl.when(ch + 1 < NCHUNK)
            def _():
                wait_idx(1 - b)
                start_rows(ch + 1, 1 - b)

            av, bv, mv = avs[b], bvs[b], mvs[b]

            def crow(r, inner):
                for g in range(D // L):
                    sl = pl.ds(g * L, L)
                    v = av[r, sl] + bv[r, sl] + mv[r, sl]
                    mv[r, sl] = jnp.maximum(v, 0.0)
                return inner
            lax.fori_loop(0, K, crow, 0)

            pltpu.sync_copy(mv, accum.at[dis[b]], add=True)

            @pl.when(ch + 2 < NCHUNK)
            def _():
                start_idx(ch + 2, b)
        return carry
    lax.fori_loop(0, NCHUNK // 2, epair, 0)

    plsc.subcore_barrier()

    def wchunk(j, carry):
        ch = s + j * NS

        @pl.when(ch < NRC)
        def _():
            pltpu.sync_copy(accum.at[pl.ds(ch * RCH, RCH)],
                            out_hbm.at[c, pl.ds(ch * RCH, RCH)])
        return carry
    lax.fori_loop(0, ZITER, wchunk, 0)


@functools.cache
def _get_sc_edge():
    return functools.partial(
        pl.kernel,
        mesh=plsc.VectorSubcoreMesh(core_axis_name="c", subcore_axis_name="s",
                                    num_cores=NC, num_subcores=NS),
        out_type=jax.ShapeDtypeStruct((NC, NN, D), jnp.float32),
        scratch_types=(
            [pltpu.VMEM((K,), jnp.int32)] * 4
            + [pltpu.VMEM((K, D), jnp.float32)] * 6
            + [pltpu.VMEM_SHARED((NN, D), jnp.float32)]
            + [pltpu.SemaphoreType.DMA] * 10
        ),
    )(_sc_edge_body)


BN = 1000
BEDG = 4000


def _dot(a, b):
    return jnp.dot(a, b, preferred_element_type=jnp.float32)


def _proj_body(h_ref, ws_ref, wd_ref, bm_ref, a_ref, b_ref):
    h = h_ref[...]
    a_ref[...] = _dot(h, ws_ref[...])
    b_ref[...] = _dot(h, wd_ref[...]) + bm_ref[...]


_proj = pl.pallas_call(
    _proj_body,
    grid=(NN // BN,),
    in_specs=[
        pl.BlockSpec((BN, D), lambda i: (i, 0)),
        pl.BlockSpec((D, D), lambda i: (0, 0)),
        pl.BlockSpec((D, D), lambda i: (0, 0)),
        pl.BlockSpec((1, D), lambda i: (0, 0)),
    ],
    out_specs=[pl.BlockSpec((BN, D), lambda i: (i, 0)),
               pl.BlockSpec((BN, D), lambda i: (i, 0))],
    out_shape=[jax.ShapeDtypeStruct((NN, D), jnp.float32),
               jax.ShapeDtypeStruct((NN, D), jnp.float32)],
)


def _edgeproj_body(ea_ref, w0_ref, w1_ref, c0_ref, c1_ref):
    ea = ea_ref[...]
    c0_ref[...] = _dot(ea, w0_ref[...])
    c1_ref[...] = _dot(ea, w1_ref[...])


_edgeproj = pl.pallas_call(
    _edgeproj_body,
    grid=(EE // BEDG,),
    in_specs=[
        pl.BlockSpec((BEDG, 16), lambda i: (i, 0)),
        pl.BlockSpec((16, D), lambda i: (0, 0)),
        pl.BlockSpec((16, D), lambda i: (0, 0)),
    ],
    out_specs=[pl.BlockSpec((BEDG, D), lambda i: (i, 0)),
               pl.BlockSpec((BEDG, D), lambda i: (i, 0))],
    out_shape=[jax.ShapeDtypeStruct((EE, D), jnp.float32),
               jax.ShapeDtypeStruct((EE, D), jnp.float32)],
)


def _upd_body(h_ref, p_ref, wuh_ref, wua_ref, bu_ref, ws_ref, wd_ref, bm_ref,
              h1_ref, a1_ref, b1_ref):
    agg = p_ref[0] + p_ref[1]
    h1 = jnp.maximum(
        _dot(h_ref[...], wuh_ref[...]) + _dot(agg, wua_ref[...]) + bu_ref[...],
        0.0)
    h1_ref[...] = h1
    a1_ref[...] = _dot(h1, ws_ref[...])
    b1_ref[...] = _dot(h1, wd_ref[...]) + bm_ref[...]


_upd = pl.pallas_call(
    _upd_body,
    grid=(NN // BN,),
    in_specs=[
        pl.BlockSpec((BN, D), lambda i: (i, 0)),
        pl.BlockSpec((NC, BN, D), lambda i: (0, i, 0)),
        pl.BlockSpec((D, D), lambda i: (0, 0)),
        pl.BlockSpec((D, D), lambda i: (0, 0)),
        pl.BlockSpec((1, D), lambda i: (0, 0)),
        pl.BlockSpec((D, D), lambda i: (0, 0)),
        pl.BlockSpec((D, D), lambda i: (0, 0)),
        pl.BlockSpec((1, D), lambda i: (0, 0)),
    ],
    out_specs=[pl.BlockSpec((BN, D), lambda i: (i, 0)),
               pl.BlockSpec((BN, D), lambda i: (i, 0)),
               pl.BlockSpec((BN, D), lambda i: (i, 0))],
    out_shape=[jax.ShapeDtypeStruct((NN, D), jnp.float32),
               jax.ShapeDtypeStruct((NN, D), jnp.float32),
               jax.ShapeDtypeStruct((NN, D), jnp.float32)],
)


def _fin_body(h_ref, p_ref, wuh_ref, wua_ref, bu_ref, o_ref):
    agg = p_ref[0] + p_ref[1]
    o_ref[...] = (_dot(h_ref[...], wuh_ref[...]) + _dot(agg, wua_ref[...])
                  + bu_ref[...])


_fin = pl.pallas_call(
    _fin_body,
    grid=(NN // BN,),
    in_specs=[
        pl.BlockSpec((BN, D), lambda i: (i, 0)),
        pl.BlockSpec((NC, BN, D), lambda i: (0, i, 0)),
        pl.BlockSpec((D, D), lambda i: (0, 0)),
        pl.BlockSpec((D, D), lambda i: (0, 0)),
        pl.BlockSpec((1, D), lambda i: (0, 0)),
    ],
    out_specs=pl.BlockSpec((BN, D), lambda i: (i, 0)),
    out_shape=jax.ShapeDtypeStruct((NN, D), jnp.float32),
)


def kernel(x, edge_index, edge_attr, Wm0, bm0, Wu0, bu0, Wm1, bm1, Wu1, bu1):
    h0 = x[:, :, 0]
    src = edge_index[0].reshape(NW, NCHUNK, K)
    dst = edge_index[1].reshape(NW, NCHUNK, K)

    bm0r = bm0.reshape(1, D)
    bm1r = bm1.reshape(1, D)
    bu0r = bu0.reshape(1, D)
    bu1r = bu1.reshape(1, D)

    _sc_edge = _get_sc_edge()
    A0, B0 = _proj(h0, Wm0[:D], Wm0[D:2 * D], bm0r)
    C0, C1 = _edgeproj(edge_attr, Wm0[2 * D:], Wm1[2 * D:])
    prt0 = _sc_edge(A0, B0, C0, src, dst)
    h1, A1, B1 = _upd(h0, prt0, Wu0[:D], Wu0[D:], bu0r,
                      Wm1[:D], Wm1[D:2 * D], bm1r)
    prt1 = _sc_edge(A1, B1, C1, src, dst)
    out = _fin(h1, prt1, Wu1[:D], Wu1[D:], bu1r)
    return out[:, :, None]
